# baseline (speedup 1.0000x reference)
import functools

import jax
import jax.numpy as jnp
from jax import lax
from jax.experimental import pallas as pl
from jax.experimental.pallas import tpu as pltpu
from jax.experimental.pallas import tpu_sc as plsc

N = 10000
E = 320000
D_IN = 128
WIDTH = 128
D_OUT = 64
LS = 8.0

NP = 10240
NC = 2
NS = 16
NW = NC * NS
L = 16

ES = E // NW
CH1 = 2000
CH2 = 8000

_SC_MESH = dict(core_axis_name="c", subcore_axis_name="s", num_cores=NC,
                num_subcores=NS)
_SC_PARAMS = pltpu.CompilerParams(needs_layout_passes=False)


def _zero_ref(ref, nwords):
    zero = jnp.zeros((L,), jnp.float32)

    def body(i, _):
        ref[pl.ds(i * L, L)] = zero
        return 0

    lax.fori_loop(0, nwords // L, body, 0)


def _sc1_body(bl0, bl1, bl2, bl3, src_hbm, dst_hbm, ew_hbm, degp, b0_v, b1_v,
              b2_v, b3_v, src_v, dst_v, ew_v, do_v, di_v):
    cid = lax.axis_index("c")
    sid = lax.axis_index("s")
    wid = sid * NC + cid
    base = wid * ES

    _zero_ref(do_v, NP)
    _zero_ref(di_v, NP)

    def chunk(k, _):
        off = base + k * CH1
        for bl, bv in ((bl0, b0_v), (bl1, b1_v), (bl2, b2_v), (bl3, b3_v)):
            pltpu.sync_copy(bl.at[pl.ds(off, CH1)], bv)
        pltpu.sync_copy(src_hbm.at[pl.ds(off, CH1)], src_v)
        pltpu.sync_copy(dst_hbm.at[pl.ds(off, CH1)], dst_v)

        @plsc.parallel_loop(0, CH1 // L, unroll=8)
        def vec(j):
            sl = pl.ds(j * L, L)
            b0 = b0_v[sl]
            b1 = b1_v[sl]
            b2 = b2_v[sl]
            b3 = b3_v[sl]
            r2 = b0 * b0 + b1 * b1 + b2 * b2 + b3 * b3
            ew = jnp.exp(r2 * (-1.0 / (LS * LS)))
            ew_v[sl] = ew
            s = src_v[sl]
            d = dst_v[sl]
            plsc.addupdate_scatter(do_v, [s], ew)
            plsc.addupdate_scatter(di_v, [d], ew)

        pltpu.sync_copy(ew_v, ew_hbm.at[pl.ds(off, CH1)])
        return 0

    lax.fori_loop(0, ES // CH1, chunk, 0)
    pltpu.sync_copy(do_v, degp.at[pl.ds(wid * NP, NP)])
    pltpu.sync_copy(di_v, degp.at[pl.ds((NW + wid) * NP, NP)])


def _sc1(bl0, bl1, bl2, bl3, src, dst):
    return pl.kernel(
        _sc1_body,
        out_type=(
            jax.ShapeDtypeStruct((E,), jnp.float32),
            jax.ShapeDtypeStruct((2 * NW * NP,), jnp.float32),
        ),
        mesh=plsc.VectorSubcoreMesh(**_SC_MESH),
        compiler_params=_SC_PARAMS,
        scratch_types=[
            pltpu.VMEM((CH1,), jnp.float32),
            pltpu.VMEM((CH1,), jnp.float32),
            pltpu.VMEM((CH1,), jnp.float32),
            pltpu.VMEM((CH1,), jnp.float32),
            pltpu.VMEM((CH1,), jnp.int32),
            pltpu.VMEM((CH1,), jnp.int32),
            pltpu.VMEM((CH1,), jnp.float32),
            pltpu.VMEM((NP,), jnp.float32),
            pltpu.VMEM((NP,), jnp.float32),
        ],
    )(bl0, bl1, bl2, bl3, src, dst)


def _sc2_body(z1t, src_hbm, dst_hbm, ew_hbm, aggt, z1_v, agg_v, src_a, dst_a,
              ew_a, src_b, dst_b, ew_b, sem_a, sem_b):
    cid = lax.axis_index("c")
    sid = lax.axis_index("s")
    wid = sid * NC + cid
    nch = E // CH2

    def start(k, sv, dv, wv, sem):
        off = k * CH2
        pltpu.async_copy(src_hbm.at[pl.ds(off, CH2)], sv, sem)
        pltpu.async_copy(dst_hbm.at[pl.ds(off, CH2)], dv, sem)
        pltpu.async_copy(ew_hbm.at[pl.ds(off, CH2)], wv, sem)

    def wait(sv, dv, wv, sem):
        pltpu.make_async_copy(src_hbm.at[pl.ds(0, CH2)], sv, sem).wait()
        pltpu.make_async_copy(dst_hbm.at[pl.ds(0, CH2)], dv, sem).wait()
        pltpu.make_async_copy(ew_hbm.at[pl.ds(0, CH2)], wv, sem).wait()

    def compute(sv, dv, wv):
        @plsc.parallel_loop(0, CH2 // L, unroll=8)
        def vec(j):
            sl = pl.ds(j * L, L)
            s = sv[sl]
            d = dv[sl]
            w = wv[sl]
            for p in range(2):
                gp = plsc.load_gather(z1_v, [s + p * NP])
                ev = plsc.bitcast(gp << 16, jnp.float32)
                od = plsc.bitcast(gp & jnp.int32(-65536), jnp.float32)
                plsc.addupdate_scatter(agg_v, [d + (2 * p) * NP], ev * w)
                plsc.addupdate_scatter(agg_v, [d + (2 * p + 1) * NP], od * w)

    start(0, src_a, dst_a, ew_a, sem_a)
    pltpu.sync_copy(z1t.at[pl.ds(wid * 2 * NP, 2 * NP)], z1_v)
    _zero_ref(agg_v, 4 * NP)

    def outer(g, _):
        c0 = 2 * g
        start(c0 + 1, src_b, dst_b, ew_b, sem_b)
        wait(src_a, dst_a, ew_a, sem_a)
        compute(src_a, dst_a, ew_a)

        @pl.when(c0 + 2 < nch)
        def _():
            start(c0 + 2, src_a, dst_a, ew_a, sem_a)

        wait(src_b, dst_b, ew_b, sem_b)
        compute(src_b, dst_b, ew_b)
        return 0

    lax.fori_loop(0, nch // 2, outer, 0)
    pltpu.sync_copy(agg_v, aggt.at[pl.ds(wid * 4 * NP, 4 * NP)])


def _sc2(z1t, src, dst, ew):
    return pl.kernel(
        _sc2_body,
        out_type=jax.ShapeDtypeStruct((WIDTH * NP,), jnp.float32),
        mesh=plsc.VectorSubcoreMesh(**_SC_MESH),
        compiler_params=_SC_PARAMS,
        scratch_types=[
            pltpu.VMEM((2 * NP,), jnp.int32),
            pltpu.VMEM((4 * NP,), jnp.float32),
            pltpu.VMEM((CH2,), jnp.int32),
            pltpu.VMEM((CH2,), jnp.int32),
            pltpu.VMEM((CH2,), jnp.float32),
            pltpu.VMEM((CH2,), jnp.int32),
            pltpu.VMEM((CH2,), jnp.int32),
            pltpu.VMEM((CH2,), jnp.float32),
            pltpu.SemaphoreType.DMA,
            pltpu.SemaphoreType.DMA,
        ],
    )(z1t, src, dst, ew)


def _sc3_body(nin_hbm, src_hbm, dst_hbm, ew_hbm, qparts, nin_v, src_v, dst_v,
              ew_v, q_v):
    cid = lax.axis_index("c")
    sid = lax.axis_index("s")
    wid = sid * NC + cid
    base = wid * ES

    pltpu.sync_copy(nin_hbm, nin_v)
    _zero_ref(q_v, NP)

    def chunk(k, _):
        off = base + k * CH1
        pltpu.sync_copy(src_hbm.at[pl.ds(off, CH1)], src_v)
        pltpu.sync_copy(dst_hbm.at[pl.ds(off, CH1)], dst_v)
        pltpu.sync_copy(ew_hbm.at[pl.ds(off, CH1)], ew_v)

        @plsc.parallel_loop(0, CH1 // L, unroll=8)
        def vec(j):
            sl = pl.ds(j * L, L)
            s = src_v[sl]
            d = dst_v[sl]
            w = ew_v[sl]
            ni = plsc.load_gather(nin_v, [d])
            plsc.addupdate_scatter(q_v, [s], w * ni)

        return 0

    lax.fori_loop(0, ES // CH1, chunk, 0)
    pltpu.sync_copy(q_v, qparts.at[pl.ds(wid * NP, NP)])


def _sc3(nin, src, dst, ew):
    return pl.kernel(
        _sc3_body,
        out_type=jax.ShapeDtypeStruct((NW * NP,), jnp.float32),
        mesh=plsc.VectorSubcoreMesh(**_SC_MESH),
        compiler_params=_SC_PARAMS,
        scratch_types=[
            pltpu.VMEM((NP,), jnp.float32),
            pltpu.VMEM((CH1,), jnp.int32),
            pltpu.VMEM((CH1,), jnp.int32),
            pltpu.VMEM((CH1,), jnp.float32),
            pltpu.VMEM((NP,), jnp.float32),
        ],
    )(nin, src, dst, ew)


BN = 1024


def _tc2_body(x_ref, we_ref, be_ref, degp_ref, w1_ref, z1p_ref, norms_ref):
    h = (lax.dot_general(x_ref[...], we_ref[...], (((1,), (0,)), ((), ())),
                         preferred_element_type=jnp.float32) + be_ref[...])
    deg = jnp.sum(degp_ref[...], axis=1)
    norms = jnp.where(deg > 0, lax.rsqrt(jnp.maximum(deg, 1e-12)), 0.0)
    norms_ref[...] = norms
    hs = h * norms[0][:, None]
    z1 = lax.dot_general(w1_ref[...], hs, (((0,), (1,)), ((), ())),
                         preferred_element_type=jnp.float32)
    z1r = z1.reshape(WIDTH // 2, 2, BN)
    ev = lax.bitcast_convert_type(z1r[:, 0, :].astype(jnp.bfloat16),
                                  jnp.uint16).astype(jnp.uint32)
    od = lax.bitcast_convert_type(z1r[:, 1, :].astype(jnp.bfloat16),
                                  jnp.uint16).astype(jnp.uint32)
    z1p_ref[...] = lax.bitcast_convert_type(ev | (od << 16), jnp.int32)


def _tc2(xp, W_emb, b_emb, degp, W1):
    return pl.pallas_call(
        _tc2_body,
        grid=(NP // BN,),
        in_specs=[
            pl.BlockSpec((BN, D_IN), lambda i: (i, 0)),
            pl.BlockSpec((D_IN, WIDTH), lambda i: (0, 0)),
            pl.BlockSpec((1, WIDTH), lambda i: (0, 0)),
            pl.BlockSpec((2, NW, BN), lambda i: (0, 0, i)),
            pl.BlockSpec((WIDTH, WIDTH), lambda i: (0, 0)),
        ],
        out_specs=[
            pl.BlockSpec((WIDTH // 2, BN), lambda i: (0, i)),
            pl.BlockSpec((2, BN), lambda i: (0, i)),
        ],
        out_shape=[
            jax.ShapeDtypeStruct((WIDTH // 2, NP), jnp.int32),
            jax.ShapeDtypeStruct((2, NP), jnp.float32),
        ],
    )(xp, W_emb, b_emb, degp, W1)


def _tc3_body(agg_ref, norms_ref, b1_ref, w2_ref, qparts_ref, b2_ref,
              out_ref):
    i = pl.program_id(0)
    ni = norms_ref[1:2, :]
    no = norms_ref[0:1, :]
    h1 = jnp.maximum(agg_ref[...] * ni + b1_ref[...], 0.0)
    h2s = h1 * no
    z2t = lax.dot_general(w2_ref[...], h2s, (((0,), (0,)), ((), ())),
                          preferred_element_type=jnp.float32)
    q = jnp.sum(qparts_ref[...], axis=0)
    part = lax.dot_general(q[None, :], z2t, (((1,), (1,)), ((), ())),
                           preferred_element_type=jnp.float32)

    @pl.when(i == 0)
    def _():
        out_ref[...] = jnp.zeros_like(out_ref)

    out_ref[...] += part

    @pl.when(i == NP // BN - 1)
    def _():
        out_ref[...] = out_ref[...] * (1.0 / N) + b2_ref[...]


def _tc3(aggt, norms, b1, W2, qparts, b2):
    return pl.pallas_call(
        _tc3_body,
        grid=(NP // BN,),
        in_specs=[
            pl.BlockSpec((WIDTH, BN), lambda i: (0, i)),
            pl.BlockSpec((2, BN), lambda i: (0, i)),
            pl.BlockSpec((WIDTH, 1), lambda i: (0, 0)),
            pl.BlockSpec((WIDTH, D_OUT), lambda i: (0, 0)),
            pl.BlockSpec((NW, BN), lambda i: (0, i)),
            pl.BlockSpec((1, D_OUT), lambda i: (0, 0)),
        ],
        out_specs=pl.BlockSpec((1, D_OUT), lambda i: (0, 0)),
        out_shape=jax.ShapeDtypeStruct((1, D_OUT), jnp.float32),
    )(aggt, norms, b1, W2, qparts, b2)


def kernel(x, edge_index, bondlength, W_emb, b_emb, W1, b1, W2, b2):
    src = edge_index[0]
    dst = edge_index[1]
    blt = bondlength.T
    bl0, bl1, bl2, bl3 = blt[0], blt[1], blt[2], blt[3]

    xp = jnp.pad(x, ((0, NP - N), (0, 0)))

    ew, degp = _sc1(bl0, bl1, bl2, bl3, src, dst)
    z1p, norms = _tc2(xp, W_emb, b_emb.reshape(1, WIDTH),
                      degp.reshape(2, NW, NP), W1)
    aggt = _sc2(z1p.reshape(WIDTH // 2 * NP), src, dst, ew)
    qparts = _sc3(norms[1], src, dst, ew)
    out = _tc3(aggt.reshape(WIDTH, NP), norms, b1.reshape(WIDTH, 1), W2,
               qparts.reshape(NW, NP), b2.reshape(1, D_OUT))
    return jnp.reshape(out, (D_OUT,))


# speedup vs baseline: 1.0538x; 1.0538x over previous
import functools

import jax
import jax.numpy as jnp
from jax import lax
from jax.experimental import pallas as pl
from jax.experimental.pallas import tpu as pltpu
from jax.experimental.pallas import tpu_sc as plsc

N = 10000
E = 320000
D_IN = 128
WIDTH = 128
D_OUT = 64
LS = 8.0

NP = 10240
NC = 2
NS = 16
NW = NC * NS
L = 16

ES = E // NW
CH1 = 2000
CH2 = 8000

_SC_MESH = dict(core_axis_name="c", subcore_axis_name="s", num_cores=NC,
                num_subcores=NS)
_SC_PARAMS = pltpu.CompilerParams(needs_layout_passes=False)


def _zero_ref(ref, nwords):
    zero = jnp.zeros((L,), jnp.float32)

    def body(i, _):
        ref[pl.ds(i * L, L)] = zero
        return 0

    lax.fori_loop(0, nwords // L, body, 0)


CHA = 1000


def _sc1_body(bl0, bl1, bl2, bl3, src_hbm, dst_hbm, ew_hbm, degp,
              b0a, b1a, b2a, b3a, sa, da, b0b, b1b, b2b, b3b, sb, db,
              ew_v, do_v, di_v, sem_a, sem_b):
    cid = lax.axis_index("c")
    sid = lax.axis_index("s")
    wid = sid * NC + cid
    base = wid * ES
    bls = (bl0, bl1, bl2, bl3)

    def start(k, bufs, sem):
        off = base + k * CHA
        for bl, bv in zip(bls, bufs[:4]):
            pltpu.async_copy(bl.at[pl.ds(off, CHA)], bv, sem)
        pltpu.async_copy(src_hbm.at[pl.ds(off, CHA)], bufs[4], sem)
        pltpu.async_copy(dst_hbm.at[pl.ds(off, CHA)], bufs[5], sem)

    def wait(bufs, sem):
        for bl, bv in zip(bls, bufs[:4]):
            pltpu.make_async_copy(bl.at[pl.ds(0, CHA)], bv, sem).wait()
        pltpu.make_async_copy(src_hbm.at[pl.ds(0, CHA)], bufs[4], sem).wait()
        pltpu.make_async_copy(dst_hbm.at[pl.ds(0, CHA)], bufs[5], sem).wait()

    def compute(k, bufs):
        b0_v, b1_v, b2_v, b3_v, src_v, dst_v = bufs

        @plsc.parallel_loop(0, CHA // L, unroll=8)
        def vec(j):
            sl = pl.ds(j * L, L)
            b0 = b0_v[sl]
            b1 = b1_v[sl]
            b2 = b2_v[sl]
            b3 = b3_v[sl]
            r2 = b0 * b0 + b1 * b1 + b2 * b2 + b3 * b3
            ew = jnp.exp(r2 * (-1.0 / (LS * LS)))
            ew_v[sl] = ew
            s = src_v[sl]
            d = dst_v[sl]
            plsc.addupdate_scatter(do_v, [s], ew)
            plsc.addupdate_scatter(di_v, [d], ew)

        pltpu.sync_copy(ew_v, ew_hbm.at[pl.ds(base + k * CHA, CHA)])

    bufs_a = (b0a, b1a, b2a, b3a, sa, da)
    bufs_b = (b0b, b1b, b2b, b3b, sb, db)
    nch = ES // CHA

    start(0, bufs_a, sem_a)
    _zero_ref(do_v, NP)
    _zero_ref(di_v, NP)

    def outer(g, _):
        c0 = 2 * g
        start(c0 + 1, bufs_b, sem_b)
        wait(bufs_a, sem_a)
        compute(c0, bufs_a)

        @pl.when(c0 + 2 < nch)
        def _():
            start(c0 + 2, bufs_a, sem_a)

        wait(bufs_b, sem_b)
        compute(c0 + 1, bufs_b)
        return 0

    lax.fori_loop(0, nch // 2, outer, 0)
    pltpu.sync_copy(do_v, degp.at[pl.ds(wid * NP, NP)])
    pltpu.sync_copy(di_v, degp.at[pl.ds((NW + wid) * NP, NP)])


def _sc1(bl0, bl1, bl2, bl3, src, dst):
    ebuf = lambda dt: pltpu.VMEM((CHA,), dt)
    return pl.kernel(
        _sc1_body,
        out_type=(
            jax.ShapeDtypeStruct((E,), jnp.float32),
            jax.ShapeDtypeStruct((2 * NW * NP,), jnp.float32),
        ),
        mesh=plsc.VectorSubcoreMesh(**_SC_MESH),
        compiler_params=_SC_PARAMS,
        scratch_types=[
            ebuf(jnp.float32), ebuf(jnp.float32), ebuf(jnp.float32),
            ebuf(jnp.float32), ebuf(jnp.int32), ebuf(jnp.int32),
            ebuf(jnp.float32), ebuf(jnp.float32), ebuf(jnp.float32),
            ebuf(jnp.float32), ebuf(jnp.int32), ebuf(jnp.int32),
            ebuf(jnp.float32),
            pltpu.VMEM((NP,), jnp.float32),
            pltpu.VMEM((NP,), jnp.float32),
            pltpu.SemaphoreType.DMA,
            pltpu.SemaphoreType.DMA,
        ],
    )(bl0, bl1, bl2, bl3, src, dst)


def _sc2_body(z1t, src_hbm, dst_hbm, ew_hbm, aggt, z1_v, agg_v, src_a, dst_a,
              ew_a, src_b, dst_b, ew_b, sem_a, sem_b):
    cid = lax.axis_index("c")
    sid = lax.axis_index("s")
    wid = sid * NC + cid
    nch = E // CH2

    def start(k, sv, dv, wv, sem):
        off = k * CH2
        pltpu.async_copy(src_hbm.at[pl.ds(off, CH2)], sv, sem)
        pltpu.async_copy(dst_hbm.at[pl.ds(off, CH2)], dv, sem)
        pltpu.async_copy(ew_hbm.at[pl.ds(off, CH2)], wv, sem)

    def wait(sv, dv, wv, sem):
        pltpu.make_async_copy(src_hbm.at[pl.ds(0, CH2)], sv, sem).wait()
        pltpu.make_async_copy(dst_hbm.at[pl.ds(0, CH2)], dv, sem).wait()
        pltpu.make_async_copy(ew_hbm.at[pl.ds(0, CH2)], wv, sem).wait()

    def compute(sv, dv, wv):
        @plsc.parallel_loop(0, CH2 // L, unroll=8)
        def vec(j):
            sl = pl.ds(j * L, L)
            s = sv[sl]
            d = dv[sl]
            w = wv[sl]
            for p in range(2):
                gp = plsc.load_gather(z1_v, [s + p * NP])
                ev = plsc.bitcast(gp << 16, jnp.float32)
                od = plsc.bitcast(gp & jnp.int32(-65536), jnp.float32)
                plsc.addupdate_scatter(agg_v, [d + (2 * p) * NP], ev * w)
                plsc.addupdate_scatter(agg_v, [d + (2 * p + 1) * NP], od * w)

    start(0, src_a, dst_a, ew_a, sem_a)
    pltpu.sync_copy(z1t.at[pl.ds(wid * 2 * NP, 2 * NP)], z1_v)
    _zero_ref(agg_v, 4 * NP)

    def outer(g, _):
        c0 = 2 * g
        start(c0 + 1, src_b, dst_b, ew_b, sem_b)
        wait(src_a, dst_a, ew_a, sem_a)
        compute(src_a, dst_a, ew_a)

        @pl.when(c0 + 2 < nch)
        def _():
            start(c0 + 2, src_a, dst_a, ew_a, sem_a)

        wait(src_b, dst_b, ew_b, sem_b)
        compute(src_b, dst_b, ew_b)
        return 0

    lax.fori_loop(0, nch // 2, outer, 0)
    pltpu.sync_copy(agg_v, aggt.at[pl.ds(wid * 4 * NP, 4 * NP)])


def _sc2(z1t, src, dst, ew):
    return pl.kernel(
        _sc2_body,
        out_type=jax.ShapeDtypeStruct((WIDTH * NP,), jnp.float32),
        mesh=plsc.VectorSubcoreMesh(**_SC_MESH),
        compiler_params=_SC_PARAMS,
        scratch_types=[
            pltpu.VMEM((2 * NP,), jnp.int32),
            pltpu.VMEM((4 * NP,), jnp.float32),
            pltpu.VMEM((CH2,), jnp.int32),
            pltpu.VMEM((CH2,), jnp.int32),
            pltpu.VMEM((CH2,), jnp.float32),
            pltpu.VMEM((CH2,), jnp.int32),
            pltpu.VMEM((CH2,), jnp.int32),
            pltpu.VMEM((CH2,), jnp.float32),
            pltpu.SemaphoreType.DMA,
            pltpu.SemaphoreType.DMA,
        ],
    )(z1t, src, dst, ew)


def _sc3_body(nin_hbm, src_hbm, dst_hbm, ew_hbm, qparts, nin_v, src_v, dst_v,
              ew_v, q_v):
    cid = lax.axis_index("c")
    sid = lax.axis_index("s")
    wid = sid * NC + cid
    base = wid * ES

    pltpu.sync_copy(nin_hbm, nin_v)
    _zero_ref(q_v, NP)

    def chunk(k, _):
        off = base + k * CH1
        pltpu.sync_copy(src_hbm.at[pl.ds(off, CH1)], src_v)
        pltpu.sync_copy(dst_hbm.at[pl.ds(off, CH1)], dst_v)
        pltpu.sync_copy(ew_hbm.at[pl.ds(off, CH1)], ew_v)

        @plsc.parallel_loop(0, CH1 // L, unroll=8)
        def vec(j):
            sl = pl.ds(j * L, L)
            s = src_v[sl]
            d = dst_v[sl]
            w = ew_v[sl]
            ni = plsc.load_gather(nin_v, [d])
            plsc.addupdate_scatter(q_v, [s], w * ni)

        return 0

    lax.fori_loop(0, ES // CH1, chunk, 0)
    pltpu.sync_copy(q_v, qparts.at[pl.ds(wid * NP, NP)])


def _sc3(nin, src, dst, ew):
    return pl.kernel(
        _sc3_body,
        out_type=jax.ShapeDtypeStruct((NW * NP,), jnp.float32),
        mesh=plsc.VectorSubcoreMesh(**_SC_MESH),
        compiler_params=_SC_PARAMS,
        scratch_types=[
            pltpu.VMEM((NP,), jnp.float32),
            pltpu.VMEM((CH1,), jnp.int32),
            pltpu.VMEM((CH1,), jnp.int32),
            pltpu.VMEM((CH1,), jnp.float32),
            pltpu.VMEM((NP,), jnp.float32),
        ],
    )(nin, src, dst, ew)


BN = 1024


def _tc1_body(x_ref, w_ref, b_ref, h_ref):
    h_ref[...] = (
        lax.dot_general(x_ref[...], w_ref[...], (((1,), (0,)), ((), ())),
                        preferred_element_type=jnp.float32)
        + b_ref[...]
    )


def _tc1(xp, W_emb, b_emb):
    return pl.pallas_call(
        _tc1_body,
        grid=(NP // BN,),
        in_specs=[
            pl.BlockSpec((BN, D_IN), lambda i: (i, 0)),
            pl.BlockSpec((D_IN, WIDTH), lambda i: (0, 0)),
            pl.BlockSpec((1, WIDTH), lambda i: (0, 0)),
        ],
        out_specs=pl.BlockSpec((BN, WIDTH), lambda i: (i, 0)),
        out_shape=jax.ShapeDtypeStruct((NP, WIDTH), jnp.float32),
    )(xp, W_emb, b_emb)


def _tc2_body(h_ref, degp_ref, w1_ref, z1p_ref, norms_ref):
    h = h_ref[...]
    deg = jnp.sum(degp_ref[...], axis=1)
    norms = jnp.where(deg > 0, lax.rsqrt(jnp.maximum(deg, 1e-12)), 0.0)
    norms_ref[...] = norms
    hs = h * norms[0][:, None]
    z1 = lax.dot_general(w1_ref[...], hs, (((0,), (1,)), ((), ())),
                         preferred_element_type=jnp.float32)
    z1r = z1.reshape(WIDTH // 2, 2, BN)
    ev = lax.bitcast_convert_type(z1r[:, 0, :].astype(jnp.bfloat16),
                                  jnp.uint16).astype(jnp.uint32)
    od = lax.bitcast_convert_type(z1r[:, 1, :].astype(jnp.bfloat16),
                                  jnp.uint16).astype(jnp.uint32)
    z1p_ref[...] = lax.bitcast_convert_type(ev | (od << 16), jnp.int32)


def _tc2(h, degp, W1):
    return pl.pallas_call(
        _tc2_body,
        grid=(NP // BN,),
        in_specs=[
            pl.BlockSpec((BN, WIDTH), lambda i: (i, 0)),
            pl.BlockSpec((2, NW, BN), lambda i: (0, 0, i)),
            pl.BlockSpec((WIDTH, WIDTH), lambda i: (0, 0)),
        ],
        out_specs=[
            pl.BlockSpec((WIDTH // 2, BN), lambda i: (0, i)),
            pl.BlockSpec((2, BN), lambda i: (0, i)),
        ],
        out_shape=[
            jax.ShapeDtypeStruct((WIDTH // 2, NP), jnp.int32),
            jax.ShapeDtypeStruct((2, NP), jnp.float32),
        ],
    )(h, degp, W1)


def _tc3_body(agg_ref, norms_ref, b1_ref, w2_ref, qparts_ref, b2_ref,
              out_ref):
    i = pl.program_id(0)
    ni = norms_ref[1:2, :]
    no = norms_ref[0:1, :]
    h1 = jnp.maximum(agg_ref[...] * ni + b1_ref[...], 0.0)
    h2s = h1 * no
    z2t = lax.dot_general(w2_ref[...], h2s, (((0,), (0,)), ((), ())),
                          preferred_element_type=jnp.float32)
    q = jnp.sum(qparts_ref[...], axis=0)
    part = lax.dot_general(q[None, :], z2t, (((1,), (1,)), ((), ())),
                           preferred_element_type=jnp.float32)

    @pl.when(i == 0)
    def _():
        out_ref[...] = jnp.zeros_like(out_ref)

    out_ref[...] += part

    @pl.when(i == NP // BN - 1)
    def _():
        out_ref[...] = out_ref[...] * (1.0 / N) + b2_ref[...]


def _tc3(aggt, norms, b1, W2, qparts, b2):
    return pl.pallas_call(
        _tc3_body,
        grid=(NP // BN,),
        in_specs=[
            pl.BlockSpec((WIDTH, BN), lambda i: (0, i)),
            pl.BlockSpec((2, BN), lambda i: (0, i)),
            pl.BlockSpec((WIDTH, 1), lambda i: (0, 0)),
            pl.BlockSpec((WIDTH, D_OUT), lambda i: (0, 0)),
            pl.BlockSpec((NW, BN), lambda i: (0, i)),
            pl.BlockSpec((1, D_OUT), lambda i: (0, 0)),
        ],
        out_specs=pl.BlockSpec((1, D_OUT), lambda i: (0, 0)),
        out_shape=jax.ShapeDtypeStruct((1, D_OUT), jnp.float32),
    )(aggt, norms, b1, W2, qparts, b2)


def kernel(x, edge_index, bondlength, W_emb, b_emb, W1, b1, W2, b2):
    src = edge_index[0]
    dst = edge_index[1]
    blt = bondlength.T
    bl0, bl1, bl2, bl3 = blt[0], blt[1], blt[2], blt[3]

    xp = jnp.pad(x, ((0, NP - N), (0, 0)))

    ew, degp = _sc1(bl0, bl1, bl2, bl3, src, dst)
    h = _tc1(xp, W_emb, b_emb.reshape(1, WIDTH))
    z1p, norms = _tc2(h, degp.reshape(2, NW, NP), W1)
    aggt = _sc2(z1p.reshape(WIDTH // 2 * NP), src, dst, ew)
    qparts = _sc3(norms[1], src, dst, ew)
    out = _tc3(aggt.reshape(WIDTH, NP), norms, b1.reshape(WIDTH, 1), W2,
               qparts.reshape(NW, NP), b2.reshape(1, D_OUT))
    return jnp.reshape(out, (D_OUT,))
